# Initial kernel scaffold; baseline (speedup 1.0000x reference)
import jax
import jax.numpy as jnp
from jax.experimental import pallas as pl


def kernel(nodes_batch, neigh_l2, neigh_l1, raw_features, W1, W2):
    raise NotImplementedError("write your pallas kernel here")


# speedup vs baseline: 3.7085x; 3.7085x over previous
import functools

import jax
import jax.numpy as jnp
from jax import lax
from jax.experimental import pallas as pl
from jax.experimental.pallas import tpu as pltpu
from jax.experimental.pallas import tpu_sc as plsc

N_NODES = 100000
D = 128
OUT = 128
B = 4096
S = 10
L1 = B * (S + 1)
NC, NS = 2, 16
NW = NC * NS
ROWS_PER_W = L1 // NW
C = 16
CHUNKS = ROWS_PER_W // C
IDX_PER_CHUNK = C * (S + 1)


def _sc_gather_sum(idx_hbm, table_hbm):
    mesh = plsc.VectorSubcoreMesh(core_axis_name="c", subcore_axis_name="s")

    @functools.partial(
        pl.kernel,
        mesh=mesh,
        out_type=jax.ShapeDtypeStruct((L1, 2 * D), jnp.float32),
        scratch_types=[
            pltpu.VMEM((IDX_PER_CHUNK,), jnp.int32),
            pltpu.VMEM((IDX_PER_CHUNK, D), jnp.float32),
            pltpu.VMEM((C, 2 * D), jnp.float32),
            pltpu.SemaphoreType.DMA,
        ],
    )
    def k(idx_h, table_h, out_h, idx_v, g_v, o_v, sem):
        wid = lax.axis_index("s") * NC + lax.axis_index("c")

        def body(kk, _):
            pltpu.sync_copy(idx_h.at[wid, kk], idx_v)
            cp0 = pltpu.async_copy(
                table_h.at[idx_v.at[pl.ds(0, 88)]], g_v.at[pl.ds(0, 88)], sem)
            cp1 = pltpu.async_copy(
                table_h.at[idx_v.at[pl.ds(88, 88)]], g_v.at[pl.ds(88, 88)], sem)
            cp0.wait()
            cp1.wait()
            for c in range(C):
                base = c * (S + 1)
                for v in range(D // 16):
                    sl = pl.ds(v * 16, 16)
                    o_v[c, pl.ds(v * 16, 16)] = g_v[base, sl]
                    acc = g_v[base + 1, sl]
                    for r in range(2, S + 1):
                        acc = acc + g_v[base + r, sl]
                    o_v[c, pl.ds(D + v * 16, 16)] = acc
            pltpu.sync_copy(o_v, out_h.at[pl.ds(wid * ROWS_PER_W + kk * C, C)])
            return 0

        lax.fori_loop(0, CHUNKS, body, 0)

    return k(idx_hbm, table_hbm)


def _tc_dense_body(comb_ref, w1s_ref, w1a_ref, w2s_ref, w2a_ref, out_ref):
    comb = comb_ref[...]
    self1 = comb[:, :D]
    nsum1 = comb[:, D:]
    h1 = jnp.clip(
        jnp.dot(self1, w1s_ref[...], preferred_element_type=jnp.float32)
        + jnp.dot(nsum1, w1a_ref[...], preferred_element_type=jnp.float32),
        0.0, 6.0)
    h1g = h1.reshape(-1, S + 1, OUT)
    self2 = h1g[:, 0, :]
    agg2 = h1g[:, 1, :]
    for r in range(2, S + 1):
        agg2 = agg2 + h1g[:, r, :]
    out_ref[...] = jnp.clip(
        jnp.dot(self2, w2s_ref[...], preferred_element_type=jnp.float32)
        + jnp.dot(agg2, w2a_ref[...], preferred_element_type=jnp.float32),
        0.0, 6.0)


def _tc_dense(comb, w1s, w1a, w2s, w2a):
    BLK = 256
    grid = (B // BLK,)
    return pl.pallas_call(
        _tc_dense_body,
        grid=grid,
        in_specs=[
            pl.BlockSpec((BLK * (S + 1), 2 * D), lambda i: (i, 0)),
            pl.BlockSpec((D, OUT), lambda i: (0, 0)),
            pl.BlockSpec((D, OUT), lambda i: (0, 0)),
            pl.BlockSpec((OUT, OUT), lambda i: (0, 0)),
            pl.BlockSpec((OUT, OUT), lambda i: (0, 0)),
        ],
        out_specs=pl.BlockSpec((BLK, OUT), lambda i: (i, 0)),
        out_shape=jax.ShapeDtypeStruct((B, OUT), jnp.float32),
    )(comb, w1s, w1a, w2s, w2a)


def kernel(nodes_batch, neigh_l2, neigh_l1, raw_features, W1, W2):
    nodes_l1 = jnp.concatenate(
        [nodes_batch[:, None], neigh_l2], axis=1).reshape(-1)
    idx11 = jnp.concatenate(
        [nodes_l1[:, None], neigh_l1], axis=1).astype(jnp.int32)
    idx = idx11.reshape(NW, CHUNKS, IDX_PER_CHUNK)

    comb = _sc_gather_sum(idx, raw_features)

    inv = jnp.float32(1.0 / S)
    w1s = W1[:, :D].T
    w1a = W1[:, D:].T * inv
    w2s = W2[:, :OUT].T
    w2a = W2[:, OUT:].T * inv
    return _tc_dense(comb, w1s, w1a, w2s, w2a)


# speedup vs baseline: 5.3423x; 1.4406x over previous
import functools

import jax
import jax.numpy as jnp
from jax import lax
from jax.experimental import pallas as pl
from jax.experimental.pallas import tpu as pltpu
from jax.experimental.pallas import tpu_sc as plsc

N_NODES = 100000
D = 128
OUT = 128
B = 4096
S = 10
L1 = B * (S + 1)
NC, NS = 2, 16
NW = NC * NS
ROWS_PER_W = L1 // NW
C = 8
CHUNKS = ROWS_PER_W // C
IDX_PER_CHUNK = C * (S + 1)
NBUF = 4


def _sc_gather_sum(idx_hbm, table_hbm):
    mesh = plsc.VectorSubcoreMesh(core_axis_name="c", subcore_axis_name="s")

    @functools.partial(
        pl.kernel,
        mesh=mesh,
        out_type=jax.ShapeDtypeStruct((L1, 2 * D), jnp.float32),
        scratch_types=[
            pltpu.VMEM((CHUNKS, IDX_PER_CHUNK), jnp.int32),
        ] + [pltpu.VMEM((IDX_PER_CHUNK, D), jnp.float32)] * NBUF
          + [pltpu.VMEM((C, 2 * D), jnp.float32)] * NBUF
          + [pltpu.SemaphoreType.DMA] * (2 * NBUF),
    )
    def k(idx_h, table_h, out_h, idx_all, g0, g1, g2, g3, o0, o1, o2, o3,
          sg0, sg1, sg2, sg3, so0, so1, so2, so3):
        g = [g0, g1, g2, g3]
        o = [o0, o1, o2, o3]
        sg = [sg0, sg1, sg2, sg3]
        so = [so0, so1, so2, so3]
        wid = lax.axis_index("s") * NC + lax.axis_index("c")

        def gather_start(c, gb, sb):
            pltpu.async_copy(table_h.at[idx_all.at[c]], gb, sb)

        def gather_wait(gb, sb):
            pltpu.make_async_copy(table_h.at[idx_all.at[0]], gb, sb).wait()

        def out_start(c, ob, sb):
            row = wid * ROWS_PER_W + c * C
            pltpu.async_copy(ob, out_h.at[pl.ds(row, C)], sb)

        def out_wait(ob, sb):
            pltpu.make_async_copy(ob, out_h.at[pl.ds(0, C)], sb).wait()

        def compute(gb, ob):
            for c in range(C):
                base = c * (S + 1)
                for v in range(D // 16):
                    sl = pl.ds(v * 16, 16)
                    ob[c, pl.ds(v * 16, 16)] = gb[base, sl]
                    acc = gb[base + 1, sl]
                    for r in range(2, S + 1):
                        acc = acc + gb[base + r, sl]
                    ob[c, pl.ds(D + v * 16, 16)] = acc

        pltpu.sync_copy(idx_h.at[wid], idx_all)
        for b in range(NBUF):
            gather_start(b, g[b], sg[b])

        def body(kk, _):
            for b in range(NBUF):
                c = kk * NBUF + b
                gather_wait(g[b], sg[b])

                @pl.when(c >= NBUF)
                def _():
                    out_wait(o[b], so[b])

                compute(g[b], o[b])
                out_start(c, o[b], so[b])

                @pl.when(c + NBUF < CHUNKS)
                def _():
                    gather_start(c + NBUF, g[b], sg[b])

            return 0

        lax.fori_loop(0, CHUNKS // NBUF, body, 0)
        for b in range(NBUF):
            out_wait(o[b], so[b])

    return k(idx_hbm, table_hbm)


def _tc_dense_body(comb_ref, w1s_ref, w1a_ref, w2s_ref, w2a_ref, out_ref):
    comb = comb_ref[...]
    self1 = comb[:, :D]
    nsum1 = comb[:, D:]
    h1 = jnp.clip(
        jnp.dot(self1, w1s_ref[...], preferred_element_type=jnp.float32)
        + jnp.dot(nsum1, w1a_ref[...], preferred_element_type=jnp.float32),
        0.0, 6.0)
    h1g = h1.reshape(-1, S + 1, OUT)
    self2 = h1g[:, 0, :]
    agg2 = h1g[:, 1, :]
    for r in range(2, S + 1):
        agg2 = agg2 + h1g[:, r, :]
    out_ref[...] = jnp.clip(
        jnp.dot(self2, w2s_ref[...], preferred_element_type=jnp.float32)
        + jnp.dot(agg2, w2a_ref[...], preferred_element_type=jnp.float32),
        0.0, 6.0)


def _tc_dense(comb, w1s, w1a, w2s, w2a):
    BLK = 256
    grid = (B // BLK,)
    return pl.pallas_call(
        _tc_dense_body,
        grid=grid,
        in_specs=[
            pl.BlockSpec((BLK * (S + 1), 2 * D), lambda i: (i, 0)),
            pl.BlockSpec((D, OUT), lambda i: (0, 0)),
            pl.BlockSpec((D, OUT), lambda i: (0, 0)),
            pl.BlockSpec((OUT, OUT), lambda i: (0, 0)),
            pl.BlockSpec((OUT, OUT), lambda i: (0, 0)),
        ],
        out_specs=pl.BlockSpec((BLK, OUT), lambda i: (i, 0)),
        out_shape=jax.ShapeDtypeStruct((B, OUT), jnp.float32),
    )(comb, w1s, w1a, w2s, w2a)


def kernel(nodes_batch, neigh_l2, neigh_l1, raw_features, W1, W2):
    nodes_l1 = jnp.concatenate(
        [nodes_batch[:, None], neigh_l2], axis=1).reshape(-1)
    idx11 = jnp.concatenate(
        [nodes_l1[:, None], neigh_l1], axis=1).astype(jnp.int32)
    idx = idx11.reshape(NW, CHUNKS, IDX_PER_CHUNK)

    comb = _sc_gather_sum(idx, raw_features)

    inv = jnp.float32(1.0 / S)
    w1s = W1[:, :D].T
    w1a = W1[:, D:].T * inv
    w2s = W2[:, :OUT].T
    w2a = W2[:, OUT:].T * inv
    return _tc_dense(comb, w1s, w1a, w2s, w2a)


# speedup vs baseline: 11.3407x; 2.1228x over previous
import functools

import jax
import jax.numpy as jnp
from jax import lax
from jax.experimental import pallas as pl
from jax.experimental.pallas import tpu as pltpu
from jax.experimental.pallas import tpu_sc as plsc

N_NODES = 100000
D = 128
OUT = 128
B = 4096
S = 10
L1 = B * (S + 1)
NC, NS = 2, 16
NW = NC * NS
ROWS_PER_W = L1 // NW
C = 32
CHUNKS = ROWS_PER_W // C
NBUF = 4
DEPTH = 3


def _sc_gather_sum(idx_hbm, table_hbm):
    mesh = plsc.VectorSubcoreMesh(core_axis_name="c", subcore_axis_name="s")

    @functools.partial(
        pl.kernel,
        mesh=mesh,
        out_type=(jax.ShapeDtypeStruct((L1, D), jnp.float32),
                  jax.ShapeDtypeStruct((L1, D), jnp.float32)),
        scratch_types=[
            pltpu.VMEM((CHUNKS, S + 1, C), jnp.int32),
        ] + [pltpu.VMEM((C, D), jnp.float32)] * NBUF
          + [pltpu.VMEM((C, D), jnp.float32)] * NBUF
          + [pltpu.SemaphoreType.DMA] * (2 * NBUF),
    )
    def k(idx_h, table_h, self_h, agg_h, idx_all,
          a0, a1, a2, a3, f0, f1, f2, f3,
          sg0, sg1, sg2, sg3, so0, so1, so2, so3):
        acc = [a0, a1, a2, a3]
        slf = [f0, f1, f2, f3]
        sg = [sg0, sg1, sg2, sg3]
        so = [so0, so1, so2, so3]
        wid = lax.axis_index("s") * NC + lax.axis_index("c")
        zeros16 = jnp.zeros((16,), jnp.float32)

        def chunk_start(c, b):
            for i in range(C):
                for v in range(D // 16):
                    acc[b][i, pl.ds(v * 16, 16)] = zeros16
            pltpu.async_copy(table_h.at[idx_all.at[c, 0]], slf[b], sg[b])
            for j in range(1, S + 1):
                pltpu.async_copy(table_h.at[idx_all.at[c, j]], acc[b], sg[b],
                                 add=True)

        def chunk_wait(b):
            for _ in range(S + 1):
                pltpu.make_async_copy(
                    table_h.at[idx_all.at[0, 0]], acc[b], sg[b]).wait()

        def out_start(c, b):
            row = wid * ROWS_PER_W + c * C
            pltpu.async_copy(slf[b], self_h.at[pl.ds(row, C)], so[b])
            pltpu.async_copy(acc[b], agg_h.at[pl.ds(row, C)], so[b])

        def out_wait(b):
            for _ in range(2):
                pltpu.make_async_copy(acc[b], agg_h.at[pl.ds(0, C)],
                                      so[b]).wait()

        pltpu.sync_copy(idx_h.at[wid], idx_all)
        for c in range(DEPTH):
            chunk_start(c, c)

        def body(kk, _):
            for bu in range(NBUF):
                c = kk * NBUF + bu
                chunk_wait(bu)
                out_start(c, bu)
                c2 = c + DEPTH
                b2 = (bu + DEPTH) % NBUF

                @pl.when(c2 < CHUNKS)
                def _():
                    @pl.when(c2 >= NBUF)
                    def _():
                        out_wait(b2)

                    chunk_start(c2, b2)

            return 0

        lax.fori_loop(0, CHUNKS // NBUF, body, 0)
        for b in range(NBUF):
            out_wait(b)

    return k(idx_hbm, table_hbm)


def _tc_dense_body(self_ref, nsum_ref, w1s_ref, w1a_ref, w2s_ref, w2a_ref,
                   out_ref):
    h1 = jnp.clip(
        jnp.dot(self_ref[...], w1s_ref[...], preferred_element_type=jnp.float32)
        + jnp.dot(nsum_ref[...], w1a_ref[...],
                  preferred_element_type=jnp.float32),
        0.0, 6.0)
    h1g = h1.reshape(-1, S + 1, OUT)
    self2 = h1g[:, 0, :]
    agg2 = h1g[:, 1, :]
    for r in range(2, S + 1):
        agg2 = agg2 + h1g[:, r, :]
    out_ref[...] = jnp.clip(
        jnp.dot(self2, w2s_ref[...], preferred_element_type=jnp.float32)
        + jnp.dot(agg2, w2a_ref[...], preferred_element_type=jnp.float32),
        0.0, 6.0)


def _tc_dense(self_rows, nsum_rows, w1s, w1a, w2s, w2a):
    BLK = 256
    grid = (B // BLK,)
    return pl.pallas_call(
        _tc_dense_body,
        grid=grid,
        in_specs=[
            pl.BlockSpec((BLK * (S + 1), D), lambda i: (i, 0)),
            pl.BlockSpec((BLK * (S + 1), D), lambda i: (i, 0)),
            pl.BlockSpec((D, OUT), lambda i: (0, 0)),
            pl.BlockSpec((D, OUT), lambda i: (0, 0)),
            pl.BlockSpec((OUT, OUT), lambda i: (0, 0)),
            pl.BlockSpec((OUT, OUT), lambda i: (0, 0)),
        ],
        out_specs=pl.BlockSpec((BLK, OUT), lambda i: (i, 0)),
        out_shape=jax.ShapeDtypeStruct((B, OUT), jnp.float32),
    )(self_rows, nsum_rows, w1s, w1a, w2s, w2a)


def kernel(nodes_batch, neigh_l2, neigh_l1, raw_features, W1, W2):
    nodes_l1 = jnp.concatenate(
        [nodes_batch[:, None], neigh_l2], axis=1).reshape(-1)
    idx11 = jnp.concatenate(
        [nodes_l1[:, None], neigh_l1], axis=1).astype(jnp.int32)
    idx = idx11.reshape(NW, CHUNKS, C, S + 1).transpose(0, 1, 3, 2)

    self_rows, nsum_rows = _sc_gather_sum(idx, raw_features)

    inv = jnp.float32(1.0 / S)
    w1s = W1[:, :D].T
    w1a = W1[:, D:].T * inv
    w2s = W2[:, :OUT].T
    w2a = W2[:, OUT:].T * inv
    return _tc_dense(self_rows, nsum_rows, w1s, w1a, w2s, w2a)
